# Initial kernel scaffold; baseline (speedup 1.0000x reference)
import jax
import jax.numpy as jnp
from jax.experimental import pallas as pl


def kernel(x, atom_types, U3, U2, U1, W3, W2, W1):
    raise NotImplementedError("write your pallas kernel here")


# speedup vs baseline: 1.3598x; 1.3598x over previous
import functools

import jax
import jax.numpy as jnp
from jax.experimental import pallas as pl


def _body(nl, a_dim, k3, k2, e_dim, prec,
          x_ref, ohx_ref, m3_ref, m2_ref, m1_ref, w3_ref, w2_ref, w1_ref,
          out_ref):
    xb = x_ref[...]
    m = xb.shape[1]
    y = (xb[:, None, :] * xb[None, :, :]).reshape(nl * nl, m)

    dn = (((1,), (0,)), ((), ()))
    q3 = jax.lax.dot_general(m3_ref[...], y, dn,
                             precision=prec, preferred_element_type=jnp.float32)
    q2 = jax.lax.dot_general(m2_ref[...], xb, dn,
                             precision=prec, preferred_element_type=jnp.float32)
    q1 = jax.lax.dot_general(m1_ref[...], xb, dn,
                             precision=prec, preferred_element_type=jnp.float32)

    w3s = w3_ref[0] * ohx_ref[0:1, :]
    w2s = w2_ref[0] * ohx_ref[0:1, :]
    w1s = w1_ref[0] * ohx_ref[0:1, :]
    for e in range(1, e_dim):
        me = ohx_ref[e:e + 1, :]
        w3s = w3s + w3_ref[e] * me
        w2s = w2s + w2_ref[e] * me
        w1s = w1s + w1_ref[e] * me

    rows = []
    for a in range(a_dim):
        acc = None
        for k in range(k3):
            r = a * k3 + k
            term = q3[r * nl:(r + 1) * nl, :] * w3s[r:r + 1, :]
            acc = term if acc is None else acc + term
        for k in range(k2):
            r = a * k2 + k
            acc = acc + q2[r * nl:(r + 1) * nl, :] * w2s[r:r + 1, :]
        outa = jnp.sum(acc * xb, axis=0, keepdims=True)
        outa = outa + q1[a:a + 1, :] * w1s[a:a + 1, :]
        rows.append(outa)
    out_ref[...] = jnp.concatenate(rows, axis=0)


def kernel(x, atom_types, U3, U2, U1, W3, W2, W1):
    n, nl, c = x.shape
    a_dim, _, _, _, k3 = U3.shape
    k2 = U2.shape[-1]
    k1 = U1.shape[-1]
    e_dim = W3.shape[1]

    b_atoms = 128
    m = b_atoms * c

    m3 = U3.transpose(0, 4, 1, 2, 3).reshape(a_dim * k3 * nl, nl * nl)
    m2 = U2.transpose(0, 3, 1, 2).reshape(a_dim * k2 * nl, nl)
    m1 = U1.transpose(0, 2, 1).reshape(a_dim * k1, nl)

    w3t = jnp.tile(W3.transpose(1, 0, 2, 3).reshape(e_dim, a_dim * k3, c), (1, 1, b_atoms))
    w2t = jnp.tile(W2.transpose(1, 0, 2, 3).reshape(e_dim, a_dim * k2, c), (1, 1, b_atoms))
    w1t = jnp.tile(W1.transpose(1, 0, 2, 3).reshape(e_dim, a_dim * k1, c), (1, 1, b_atoms))

    oh = (atom_types[None, :] == jnp.arange(e_dim, dtype=atom_types.dtype)[:, None])
    ohx = jnp.repeat(oh.astype(x.dtype), c, axis=1)

    x2 = x.transpose(1, 0, 2).reshape(nl, n * c)

    body = functools.partial(_body, nl, a_dim, k3, k2, e_dim,
                             jax.lax.Precision.HIGHEST)
    out = pl.pallas_call(
        body,
        grid=((n * c) // m,),
        in_specs=[
            pl.BlockSpec((nl, m), lambda i: (0, i)),
            pl.BlockSpec((e_dim, m), lambda i: (0, i)),
            pl.BlockSpec(m3.shape, lambda i: (0, 0)),
            pl.BlockSpec(m2.shape, lambda i: (0, 0)),
            pl.BlockSpec(m1.shape, lambda i: (0, 0)),
            pl.BlockSpec(w3t.shape, lambda i: (0, 0, 0)),
            pl.BlockSpec(w2t.shape, lambda i: (0, 0, 0)),
            pl.BlockSpec(w1t.shape, lambda i: (0, 0, 0)),
        ],
        out_specs=pl.BlockSpec((a_dim, m), lambda i: (0, i)),
        out_shape=jax.ShapeDtypeStruct((a_dim, n * c), x.dtype),
    )(x2, ohx, m3, m2, m1, w3t, w2t, w1t)
    return out.reshape(a_dim, n, c).transpose(1, 0, 2)


# speedup vs baseline: 3.2256x; 2.3721x over previous
import functools

import jax
import jax.numpy as jnp
from jax.experimental import pallas as pl


def _body(nl, a_dim, k3, k2, e_dim, prec,
          x_ref, ohx_ref, m3_ref, m2_ref, m1_ref, w3_ref, w2_ref, w1_ref,
          out_ref):
    xb = x_ref[...]
    m = xb.shape[1]
    y = (xb[:, None, :] * xb[None, :, :]).reshape(nl * nl, m)

    dn = (((1,), (0,)), ((), ()))
    q3 = jax.lax.dot_general(m3_ref[...], y, dn,
                             precision=prec, preferred_element_type=jnp.float32)
    q2 = jax.lax.dot_general(m2_ref[...], xb, dn,
                             precision=prec, preferred_element_type=jnp.float32)
    q1 = jax.lax.dot_general(m1_ref[...], xb, dn,
                             precision=prec, preferred_element_type=jnp.float32)

    w3s = w3_ref[0] * ohx_ref[0:1, :]
    w2s = w2_ref[0] * ohx_ref[0:1, :]
    w1s = w1_ref[0] * ohx_ref[0:1, :]
    for e in range(1, e_dim):
        me = ohx_ref[e:e + 1, :]
        w3s = w3s + w3_ref[e] * me
        w2s = w2s + w2_ref[e] * me
        w1s = w1s + w1_ref[e] * me

    rows = []
    for a in range(a_dim):
        acc = None
        for k in range(k3):
            r = a * k3 + k
            term = q3[r * nl:(r + 1) * nl, :] * w3s[r:r + 1, :]
            acc = term if acc is None else acc + term
        for k in range(k2):
            r = a * k2 + k
            acc = acc + q2[r * nl:(r + 1) * nl, :] * w2s[r:r + 1, :]
        outa = jnp.sum(acc * xb, axis=0, keepdims=True)
        outa = outa + q1[a:a + 1, :] * w1s[a:a + 1, :]
        rows.append(outa)
    out_ref[...] = jnp.concatenate(rows, axis=0)


def kernel(x, atom_types, U3, U2, U1, W3, W2, W1):
    n, nl, c = x.shape
    a_dim, _, _, _, k3 = U3.shape
    k2 = U2.shape[-1]
    k1 = U1.shape[-1]
    e_dim = W3.shape[1]

    b_atoms = 128
    m = b_atoms * c

    m3 = U3.transpose(0, 4, 1, 2, 3).reshape(a_dim * k3 * nl, nl * nl)
    m2 = U2.transpose(0, 3, 1, 2).reshape(a_dim * k2 * nl, nl)
    m1 = U1.transpose(0, 2, 1).reshape(a_dim * k1, nl)

    w3t = jnp.tile(W3.transpose(1, 0, 2, 3).reshape(e_dim, a_dim * k3, c), (1, 1, b_atoms))
    w2t = jnp.tile(W2.transpose(1, 0, 2, 3).reshape(e_dim, a_dim * k2, c), (1, 1, b_atoms))
    w1t = jnp.tile(W1.transpose(1, 0, 2, 3).reshape(e_dim, a_dim * k1, c), (1, 1, b_atoms))

    oh = (atom_types[None, :] == jnp.arange(e_dim, dtype=atom_types.dtype)[:, None])
    ohx = jnp.repeat(oh.astype(x.dtype), c, axis=1)

    x2 = x.transpose(1, 0, 2).reshape(nl, n * c)

    body = functools.partial(_body, nl, a_dim, k3, k2, e_dim,
                             jax.lax.Precision.DEFAULT)
    out = pl.pallas_call(
        body,
        grid=((n * c) // m,),
        in_specs=[
            pl.BlockSpec((nl, m), lambda i: (0, i)),
            pl.BlockSpec((e_dim, m), lambda i: (0, i)),
            pl.BlockSpec(m3.shape, lambda i: (0, 0)),
            pl.BlockSpec(m2.shape, lambda i: (0, 0)),
            pl.BlockSpec(m1.shape, lambda i: (0, 0)),
            pl.BlockSpec(w3t.shape, lambda i: (0, 0, 0)),
            pl.BlockSpec(w2t.shape, lambda i: (0, 0, 0)),
            pl.BlockSpec(w1t.shape, lambda i: (0, 0, 0)),
        ],
        out_specs=pl.BlockSpec((a_dim, m), lambda i: (0, i)),
        out_shape=jax.ShapeDtypeStruct((a_dim, n * c), x.dtype),
    )(x2, ohx, m3, m2, m1, w3t, w2t, w1t)
    return out.reshape(a_dim, n, c).transpose(1, 0, 2)


# speedup vs baseline: 3.4094x; 1.0570x over previous
import functools

import jax
import jax.numpy as jnp
from jax.experimental import pallas as pl

_HALF = 8


def _body(nl, a_dim, k3, k2, e_dim, r3, r2, prec,
          x_ref, ohx_ref, mf_ref, w3_ref, w2_ref, w1_ref, out_ref):
    xb = x_ref[...]

    pieces = [xb[j:j + 1, :] * xb for j in range(_HALF)]
    pieces += [xb[j:j + 1, :] * xb[_HALF:, :] for j in range(_HALF, nl)]
    pieces.append(xb)
    ycat = jnp.concatenate(pieces, axis=0)

    q = jax.lax.dot_general(mf_ref[...], ycat, (((1,), (0,)), ((), ())),
                            precision=prec, preferred_element_type=jnp.float32)
    q3 = q[:r3, :]
    q2 = q[r3:r3 + r2, :]
    q1 = q[r3 + r2:, :]

    w3s = w3_ref[0] * ohx_ref[0:1, :]
    w2s = w2_ref[0] * ohx_ref[0:1, :]
    w1s = w1_ref[0] * ohx_ref[0:1, :]
    for e in range(1, e_dim):
        me = ohx_ref[e:e + 1, :]
        w3s = w3s + w3_ref[e] * me
        w2s = w2s + w2_ref[e] * me
        w1s = w1s + w1_ref[e] * me

    rows = []
    for a in range(a_dim):
        acc = None
        for k in range(k3):
            r = a * k3 + k
            term = q3[r * nl:(r + 1) * nl, :] * w3s[r:r + 1, :]
            acc = term if acc is None else acc + term
        for k in range(k2):
            r = a * k2 + k
            acc = acc + q2[r * nl:(r + 1) * nl, :] * w2s[r:r + 1, :]
        outa = jnp.sum(acc * xb, axis=0, keepdims=True)
        outa = outa + q1[a:a + 1, :] * w1s[a:a + 1, :]
        rows.append(outa)
    out_ref[...] = jnp.concatenate(rows, axis=0)


def kernel(x, atom_types, U3, U2, U1, W3, W2, W1):
    n, nl, c = x.shape
    a_dim, _, _, _, k3 = U3.shape
    k2 = U2.shape[-1]
    k1 = U1.shape[-1]
    e_dim = W3.shape[1]

    b_atoms = 128
    m = b_atoms * c

    r3, r2, r1 = a_dim * k3 * nl, a_dim * k2 * nl, a_dim * k1
    h = _HALF
    ncols_tri = h * nl + (nl - h) * (nl - h)

    m3g = U3.transpose(0, 4, 1, 2, 3).reshape(r3, nl, nl)
    fold = m3g.at[:, :h, h:].add(m3g[:, h:, :h].transpose(0, 2, 1))
    m3tri = jnp.concatenate([
        fold[:, :h, :].reshape(r3, h * nl),
        fold[:, h:, h:].reshape(r3, (nl - h) * (nl - h)),
    ], axis=1)

    m2 = U2.transpose(0, 3, 1, 2).reshape(r2, nl)
    m1 = U1.transpose(0, 2, 1).reshape(r1, nl)

    zf = jnp.zeros((r3, nl), U3.dtype)
    m_full = jnp.concatenate([
        jnp.concatenate([m3tri, zf], axis=1),
        jnp.concatenate([jnp.zeros((r2, ncols_tri), U2.dtype), m2], axis=1),
        jnp.concatenate([jnp.zeros((r1, ncols_tri), U1.dtype), m1], axis=1),
    ], axis=0)

    w3t = jnp.tile(W3.transpose(1, 0, 2, 3).reshape(e_dim, a_dim * k3, c), (1, 1, b_atoms))
    w2t = jnp.tile(W2.transpose(1, 0, 2, 3).reshape(e_dim, a_dim * k2, c), (1, 1, b_atoms))
    w1t = jnp.tile(W1.transpose(1, 0, 2, 3).reshape(e_dim, a_dim * k1, c), (1, 1, b_atoms))

    oh = (atom_types[None, :] == jnp.arange(e_dim, dtype=atom_types.dtype)[:, None])
    ohx = jnp.repeat(oh.astype(x.dtype), c, axis=1)

    x2 = x.transpose(1, 0, 2).reshape(nl, n * c)

    body = functools.partial(_body, nl, a_dim, k3, k2, e_dim, r3, r2,
                             jax.lax.Precision.DEFAULT)
    out = pl.pallas_call(
        body,
        grid=((n * c) // m,),
        in_specs=[
            pl.BlockSpec((nl, m), lambda i: (0, i)),
            pl.BlockSpec((e_dim, m), lambda i: (0, i)),
            pl.BlockSpec(m_full.shape, lambda i: (0, 0)),
            pl.BlockSpec(w3t.shape, lambda i: (0, 0, 0)),
            pl.BlockSpec(w2t.shape, lambda i: (0, 0, 0)),
            pl.BlockSpec(w1t.shape, lambda i: (0, 0, 0)),
        ],
        out_specs=pl.BlockSpec((a_dim, m), lambda i: (0, i)),
        out_shape=jax.ShapeDtypeStruct((a_dim, n * c), x.dtype),
    )(x2, ohx, m_full, w3t, w2t, w1t)
    return out.reshape(a_dim, n, c).transpose(1, 0, 2)
